# baseline (speedup 1.0000x reference)
import functools

import jax
import jax.numpy as jnp
from jax import lax
from jax.experimental import pallas as pl
from jax.experimental.pallas import tpu as pltpu
from jax.experimental.pallas import tpu_sc as plsc

_BBLK = 256


def _index_body(x_ref, t_ref, idx_ref):
    xb = x_ref[...]
    _, H, W = xb.shape
    colsum = jnp.sum(xb, axis=1)
    rowsum = jnp.sum(xb, axis=2)

    def first_argmax(a):
        n = a.shape[-1]
        m = jnp.max(a, axis=-1, keepdims=True)
        ii = lax.broadcasted_iota(jnp.int32, a.shape, 1)
        return jnp.min(jnp.where(a == m, ii, n), axis=-1)

    gx = first_argmax(colsum)
    gy = first_argmax(rowsum)
    idx_ref[...] = t_ref[...] * (H * W) + gx * W + gy


def _flat_indices(xr, t):
    B, H, W = xr.shape
    return pl.pallas_call(
        _index_body,
        grid=(B // _BBLK,),
        in_specs=[
            pl.BlockSpec((_BBLK, H, W), lambda i: (i, 0, 0)),
            pl.BlockSpec((_BBLK,), lambda i: (i,)),
        ],
        out_specs=pl.BlockSpec((_BBLK,), lambda i: (i,)),
        out_shape=jax.ShapeDtypeStruct((B,), jnp.int32),
    )(xr, t)


_ROW = 32


@functools.lru_cache(maxsize=None)
def _make_gather(NWORDS, D, B):
    V = NWORDS // _ROW
    info = plsc.get_sparse_core_info()
    NC = info.num_cores
    NW = NC * info.num_subcores
    L = info.num_lanes
    b_per_w = B // NW
    G = b_per_w // L
    mesh = plsc.VectorSubcoreMesh(core_axis_name="c", subcore_axis_name="s")

    @functools.partial(
        pl.kernel,
        mesh=mesh,
        compiler_params=pltpu.CompilerParams(
            use_tc_tiling_on_sc=False, needs_layout_passes=False),
        out_type=jax.ShapeDtypeStruct((B, D), jnp.float32),
        scratch_types=[
            pltpu.VMEM((b_per_w,), jnp.int32),
            pltpu.VMEM((b_per_w,), jnp.int32),
            pltpu.VMEM((b_per_w,), jnp.int32),
            pltpu.VMEM((b_per_w, _ROW), jnp.float32),
            pltpu.VMEM((b_per_w, _ROW), jnp.float32),
            pltpu.VMEM((b_per_w, D), jnp.float32),
            pltpu.SemaphoreType.DMA,
        ],
    )
    def k(table_hbm, idx_hbm, out_hbm, flat_v, idx0_v, idx1_v, rows0_v,
          rows1_v, out_v, sem):
        wid = lax.axis_index("s") * NC + lax.axis_index("c")
        base = wid * b_per_w
        pltpu.sync_copy(idx_hbm.at[pl.ds(base, b_per_w)], flat_v)
        for g in range(G):
            f = flat_v[pl.ds(g * L, L)]
            s = f * D
            r0 = lax.shift_right_logical(s, 5)
            r1 = jnp.minimum(r0 + 1, V - 1)
            idx0_v[pl.ds(g * L, L)] = r0
            idx1_v[pl.ds(g * L, L)] = r1
        c0 = pltpu.async_copy(table_hbm.at[idx0_v], rows0_v, sem)
        c1 = pltpu.async_copy(table_hbm.at[idx1_v], rows1_v, sem)
        c0.wait()
        c1.wait()
        for g in range(G):
            b_vec = g * L + lax.iota(jnp.int32, L)
            f = flat_v[pl.ds(g * L, L)]
            off = jnp.bitwise_and(f * D, _ROW - 1)
            for j in range(D):
                tpos = off + j
                col = jnp.bitwise_and(tpos, _ROW - 1)
                hi = tpos >= _ROW
                v0 = plsc.load_gather(rows0_v, [b_vec, col])
                v1 = plsc.load_gather(rows1_v, [b_vec, col])
                val = jnp.where(hi, v1, v0)
                plsc.store_scatter(out_v, [b_vec, jnp.full((L,), j, jnp.int32)], val)
        pltpu.sync_copy(out_v, out_hbm.at[pl.ds(base, b_per_w)])

    return k


def kernel(x, t, params):
    B, _, H, W = x.shape
    T = params.shape[0]
    NA = params.shape[-1]
    xr = x.reshape(B, H, W)
    flat = _flat_indices(xr, t.astype(jnp.int32))
    table = params.reshape(T * H * W * NA // _ROW, _ROW)
    return _make_gather(T * H * W * NA, NA, B)(table, flat)


# speedup vs baseline: 1.1332x; 1.1332x over previous
import functools

import jax
import jax.numpy as jnp
from jax import lax
from jax.experimental import pallas as pl
from jax.experimental.pallas import tpu as pltpu
from jax.experimental.pallas import tpu_sc as plsc

_BBLK = 256


def _index_body(x_ref, t_ref, idx_ref):
    xb = x_ref[...]
    H, W, _ = xb.shape
    colsum = jnp.sum(xb, axis=0)
    rowsum = jnp.sum(xb, axis=1)

    def first_argmax(a):
        n = a.shape[0]
        m = jnp.max(a, axis=0, keepdims=True)
        ii = lax.broadcasted_iota(jnp.int32, a.shape, 0)
        return jnp.min(jnp.where(a == m, ii, n), axis=0)

    gx = first_argmax(colsum)
    gy = first_argmax(rowsum)
    idx_ref[...] = t_ref[...] * (H * W) + gx * W + gy


def _flat_indices(xt, t):
    H, W, B = xt.shape
    return pl.pallas_call(
        _index_body,
        grid=(B // _BBLK,),
        in_specs=[
            pl.BlockSpec((H, W, _BBLK), lambda i: (0, 0, i)),
            pl.BlockSpec((_BBLK,), lambda i: (i,)),
        ],
        out_specs=pl.BlockSpec((_BBLK,), lambda i: (i,)),
        out_shape=jax.ShapeDtypeStruct((B,), jnp.int32),
    )(xt, t)


_SC_PARAMS = pltpu.CompilerParams(
    use_tc_tiling_on_sc=True, needs_layout_passes=False)


@functools.lru_cache(maxsize=None)
def _make_gather(H, NA, W, T, B, R):
    info = plsc.get_sparse_core_info()
    NC = info.num_cores
    NW = NC * info.num_subcores
    L = info.num_lanes
    b_per_w = B // NW
    CH = L
    mesh = plsc.VectorSubcoreMesh(core_axis_name="c", subcore_axis_name="s")

    @functools.partial(
        pl.kernel,
        mesh=mesh,
        compiler_params=_SC_PARAMS,
        out_type=jax.ShapeDtypeStruct((NW, NA, b_per_w), jnp.float32),
        scratch_types=[
            pltpu.VMEM((b_per_w,), jnp.int32),
            pltpu.VMEM((2, CH), jnp.int32),
            pltpu.VMEM((2, CH, 128), jnp.float32),
            pltpu.VMEM((NA, b_per_w), jnp.float32),
            pltpu.SemaphoreType.DMA,
        ],
    )
    def k(tab_hbm, idx_hbm, out_hbm, idx_v, rowid_v, rows_v, out_v, sem):
        wid = lax.axis_index("s") * NC + lax.axis_index("c")
        base = wid * b_per_w
        pltpu.sync_copy(idx_hbm.at[pl.ds(base, b_per_w)], idx_v)
        lanes = lax.iota(jnp.int32, L)

        def chunk(c):
            p = idx_v[pl.ds(c * CH, CH)]
            f = p * NA
            r0 = lax.shift_right_logical(f, 7)
            rowid_v[0, :] = r0
            rowid_v[1, :] = jnp.minimum(r0 + 1, R - 1)
            cp0 = pltpu.async_copy(
                tab_hbm.at[rowid_v.at[0]], rows_v.at[0], sem)
            cp1 = pltpu.async_copy(
                tab_hbm.at[rowid_v.at[1]], rows_v.at[1], sem)
            cp0.wait()
            cp1.wait()
            off = jnp.bitwise_and(f, 127)
            for a in range(NA):
                pos = off + a
                hi = lax.shift_right_logical(pos, 7)
                col = jnp.bitwise_and(pos, 127)
                val = plsc.load_gather(rows_v, [hi, lanes, col])
                out_v[a, pl.ds(c * CH, CH)] = val

        pl.loop(0, b_per_w // CH)(chunk)
        pltpu.sync_copy(out_v, out_hbm.at[wid])

    return k


def kernel(x, t, params):
    B, _, H, W = x.shape
    T = params.shape[0]
    NA = params.shape[-1]
    xt = jnp.squeeze(x.transpose(1, 2, 3, 0), axis=0)
    flat = _flat_indices(xt, t.astype(jnp.int32))
    R = T * H * W * NA // 128
    tab = params.reshape(R, 128)
    out3 = _make_gather(H, NA, W, T, B, R)(tab, flat)
    return out3.transpose(0, 2, 1).reshape(B, NA)
